# baseline (speedup 1.0000x reference)
import functools

import jax
import jax.numpy as jnp
from jax import lax
from jax.experimental import pallas as pl
from jax.experimental.pallas import tpu as pltpu
from jax.experimental.pallas import tpu_sc as plsc

NUM_CORES = 2
NUM_SUBCORES = 16
NUM_WORKERS = NUM_CORES * NUM_SUBCORES
LANES = 16

NUM_USERS = 1000
NUM_ITEMS = 1000000
BATCH = 16384
EMBED_DIM = 64
PACK = 128
B_PER_W = BATCH // NUM_WORKERS
IDX_CHUNK = 128
HALF = 256
N_HALF = B_PER_W // HALF
CHUNKS_PER_HALF = HALF // IDX_CHUNK
N_CHUNKS = N_HALF * CHUNKS_PER_HALF


def _body(items2_hbm, uoff_hbm, ioff_hbm, utp_hbm, itp_hbm, out_hbm,
          items2_v, uoff_v, ioff_v, u_tab, i_rows, out_v, sem):
    c = lax.axis_index("c")
    s = lax.axis_index("s")
    wid = s * NUM_CORES + c
    base = wid * B_PER_W

    pltpu.sync_copy(items2_hbm.at[wid], items2_v)
    pltpu.sync_copy(uoff_hbm.at[pl.ds(base, B_PER_W)], uoff_v)
    pltpu.sync_copy(ioff_hbm.at[pl.ds(base, B_PER_W)], ioff_v)
    pltpu.sync_copy(utp_hbm, u_tab)

    last_lane = lax.iota(jnp.int32, LANES) == (LANES - 1)

    for h in range(N_HALF):
        copies = []
        for j in range(CHUNKS_PER_HALF):
            jj = h * CHUNKS_PER_HALF + j
            copies.append(pltpu.async_copy(
                itp_hbm.at[items2_v.at[jj]],
                i_rows.at[pl.ds(j * IDX_CHUNK, IDX_CHUNK)], sem))
        for cp in copies:
            cp.wait()

        def pair_block(g, _, h=h):
            uo = uoff_v[pl.ds(h * HALF + g * LANES, LANES)]
            io = ioff_v[pl.ds(h * HALF + g * LANES, LANES)]
            for k in range(LANES):
                r = g * LANES + k
                ub = uo[k]
                ib = io[k]
                acc = (u_tab[pl.ds(ub, LANES)]
                       * i_rows[r, pl.ds(ib, LANES)])
                for q in range(1, EMBED_DIM // LANES):
                    acc = acc + (u_tab[pl.ds(ub + q * LANES, LANES)]
                                 * i_rows[r, pl.ds(ib + q * LANES, LANES)])
                csum = plsc.cumsum(acc)
                pos = jnp.zeros((LANES,), jnp.int32) + (h * HALF + r)
                plsc.store_scatter(out_v, [pos], csum, mask=last_lane)
            return 0

        lax.fori_loop(0, HALF // LANES, pair_block, 0)

    pltpu.sync_copy(out_v, out_hbm.at[pl.ds(base, B_PER_W)])


def kernel(users, items, user_table, item_table):
    utp = user_table.reshape(NUM_USERS * EMBED_DIM)
    itp = item_table.reshape(NUM_ITEMS // 2, PACK)
    items_r = (items >> 1).reshape(NUM_WORKERS, N_CHUNKS, IDX_CHUNK)
    uoff = users * EMBED_DIM
    ioff = (items & 1) * EMBED_DIM
    mesh = plsc.VectorSubcoreMesh(core_axis_name="c", subcore_axis_name="s")
    run = pl.kernel(
        _body,
        out_type=jax.ShapeDtypeStruct((BATCH,), jnp.float32),
        mesh=mesh,
        scratch_types=[
            pltpu.VMEM((N_CHUNKS, IDX_CHUNK), jnp.int32),
            pltpu.VMEM((B_PER_W,), jnp.int32),
            pltpu.VMEM((B_PER_W,), jnp.int32),
            pltpu.VMEM((NUM_USERS * EMBED_DIM,), jnp.float32),
            pltpu.VMEM((HALF, PACK), jnp.float32),
            pltpu.VMEM((B_PER_W,), jnp.float32),
            pltpu.SemaphoreType.DMA,
        ],
        compiler_params=pltpu.CompilerParams(needs_layout_passes=False),
    )
    return run(items_r, uoff, ioff, utp, itp)


# speedup vs baseline: 1.6905x; 1.6905x over previous
import functools

import jax
import jax.numpy as jnp
from jax import lax
from jax.experimental import pallas as pl
from jax.experimental.pallas import tpu as pltpu
from jax.experimental.pallas import tpu_sc as plsc

NUM_CORES = 2
NUM_SUBCORES = 16
NUM_WORKERS = NUM_CORES * NUM_SUBCORES
LANES = 16

NUM_USERS = 1000
NUM_ITEMS = 1000000
BATCH = 16384
EMBED_DIM = 64
B_PER_W = BATCH // NUM_WORKERS
N_GROUPS = B_PER_W // LANES


def _body(items_hbm, uoff_hbm, utf_hbm, it_hbm, out_hbm,
          items_v, uoff_v, u_tab, i_rows, out_v, sem):
    c = lax.axis_index("c")
    s = lax.axis_index("s")
    wid = s * NUM_CORES + c
    base = wid * B_PER_W

    pltpu.sync_copy(items_hbm.at[pl.ds(base, B_PER_W)], items_v)
    pltpu.sync_copy(uoff_hbm.at[pl.ds(base, B_PER_W)], uoff_v)
    pltpu.sync_copy(utf_hbm, u_tab)

    def issue(g, _):
        iv = items_v[pl.ds(g * LANES, LANES)]
        for k in range(LANES):
            r = g * LANES + k
            pltpu.async_copy(it_hbm.at[pl.ds(iv[k], 1), :],
                             i_rows.at[pl.ds(r, 1), :], sem)
        return 0

    lax.fori_loop(0, N_GROUPS, issue, 0)

    pltpu.make_async_copy(it_hbm.at[pl.ds(0, B_PER_W // 2), :],
                          i_rows.at[pl.ds(0, B_PER_W // 2), :], sem).wait()
    pltpu.make_async_copy(it_hbm.at[pl.ds(0, B_PER_W // 2), :],
                          i_rows.at[pl.ds(B_PER_W // 2, B_PER_W // 2), :], sem).wait()

    last_lane = lax.iota(jnp.int32, LANES) == (LANES - 1)

    def pair_block(g, _):
        uo = uoff_v[pl.ds(g * LANES, LANES)]
        for k in range(LANES):
            r = g * LANES + k
            ub = uo[k]
            acc = (u_tab[pl.ds(ub, LANES)]
                   * i_rows[r, pl.ds(0, LANES)])
            for q in range(1, EMBED_DIM // LANES):
                acc = acc + (u_tab[pl.ds(ub + q * LANES, LANES)]
                             * i_rows[r, pl.ds(q * LANES, LANES)])
            csum = plsc.cumsum(acc)
            pos = jnp.zeros((LANES,), jnp.int32) + r
            plsc.store_scatter(out_v, [pos], csum, mask=last_lane)
        return 0

    lax.fori_loop(0, N_GROUPS, pair_block, 0)

    pltpu.sync_copy(out_v, out_hbm.at[pl.ds(base, B_PER_W)])


def kernel(users, items, user_table, item_table):
    utf = user_table.reshape(NUM_USERS * EMBED_DIM)
    uoff = users * EMBED_DIM
    mesh = plsc.VectorSubcoreMesh(core_axis_name="c", subcore_axis_name="s")
    run = pl.kernel(
        _body,
        out_type=jax.ShapeDtypeStruct((BATCH,), jnp.float32),
        mesh=mesh,
        scratch_types=[
            pltpu.VMEM((B_PER_W,), jnp.int32),
            pltpu.VMEM((B_PER_W,), jnp.int32),
            pltpu.VMEM((NUM_USERS * EMBED_DIM,), jnp.float32),
            pltpu.VMEM((B_PER_W, EMBED_DIM), jnp.float32),
            pltpu.VMEM((B_PER_W,), jnp.float32),
            pltpu.SemaphoreType.DMA,
        ],
        compiler_params=pltpu.CompilerParams(needs_layout_passes=False),
    )
    return run(items, uoff, utf, item_table)


# speedup vs baseline: 2.5215x; 1.4916x over previous
import functools

import jax
import jax.numpy as jnp
from jax import lax
from jax.experimental import pallas as pl
from jax.experimental.pallas import tpu as pltpu
from jax.experimental.pallas import tpu_sc as plsc

NUM_CORES = 2
NUM_SUBCORES = 16
NUM_WORKERS = NUM_CORES * NUM_SUBCORES
LANES = 16

NUM_USERS = 1000
NUM_ITEMS = 1000000
BATCH = 16384
EMBED_DIM = 64
B_PER_W = BATCH // NUM_WORKERS
N_GROUPS = B_PER_W // LANES


def _body(items_hbm, uoff_hbm, utf_hbm, it_hbm, out_hbm,
          items_v, uoff_v, u_tab, i_rows, out_v, sem):
    c = lax.axis_index("c")
    s = lax.axis_index("s")
    wid = s * NUM_CORES + c
    base = wid * B_PER_W

    pltpu.sync_copy(items_hbm.at[pl.ds(base, B_PER_W)], items_v)
    pltpu.sync_copy(uoff_hbm.at[pl.ds(base, B_PER_W)], uoff_v)
    pltpu.sync_copy(utf_hbm, u_tab)

    def issue(g, _):
        iv = items_v[pl.ds(g * LANES, LANES)]
        for k in range(LANES):
            r = g * LANES + k
            pltpu.async_copy(it_hbm.at[iv[k] >> 3, pl.ds(iv[k] & 7, 1), :],
                             i_rows.at[pl.ds(r, 1), :], sem)
        return 0

    lax.fori_loop(0, N_GROUPS, issue, 0)

    for j in range(B_PER_W // 8):
        pltpu.make_async_copy(it_hbm.at[0, :, :],
                              i_rows.at[pl.ds(j * 8, 8), :], sem).wait()

    last_lane = lax.iota(jnp.int32, LANES) == (LANES - 1)

    def pair_block(g, _):
        uo = uoff_v[pl.ds(g * LANES, LANES)]
        for k in range(LANES):
            r = g * LANES + k
            ub = uo[k]
            acc = (u_tab[pl.ds(ub, LANES)]
                   * i_rows[r, pl.ds(0, LANES)])
            for q in range(1, EMBED_DIM // LANES):
                acc = acc + (u_tab[pl.ds(ub + q * LANES, LANES)]
                             * i_rows[r, pl.ds(q * LANES, LANES)])
            csum = plsc.cumsum(acc)
            pos = jnp.zeros((LANES,), jnp.int32) + r
            plsc.store_scatter(out_v, [pos], csum, mask=last_lane)
        return 0

    lax.fori_loop(0, N_GROUPS, pair_block, 0)

    pltpu.sync_copy(out_v, out_hbm.at[pl.ds(base, B_PER_W)])


def kernel(users, items, user_table, item_table):
    it3 = item_table.reshape(NUM_ITEMS // 8, 8, EMBED_DIM)
    utf = user_table.reshape(NUM_USERS * EMBED_DIM)
    uoff = users * EMBED_DIM
    mesh = plsc.VectorSubcoreMesh(core_axis_name="c", subcore_axis_name="s")
    run = pl.kernel(
        _body,
        out_type=jax.ShapeDtypeStruct((BATCH,), jnp.float32),
        mesh=mesh,
        scratch_types=[
            pltpu.VMEM((B_PER_W,), jnp.int32),
            pltpu.VMEM((B_PER_W,), jnp.int32),
            pltpu.VMEM((NUM_USERS * EMBED_DIM,), jnp.float32),
            pltpu.VMEM((B_PER_W, EMBED_DIM), jnp.float32),
            pltpu.VMEM((B_PER_W,), jnp.float32),
            pltpu.SemaphoreType.DMA,
        ],
        compiler_params=pltpu.CompilerParams(needs_layout_passes=False),
    )
    return run(items, uoff, utf, it3)
